# Initial kernel scaffold; baseline (speedup 1.0000x reference)
import jax
import jax.numpy as jnp
from jax.experimental import pallas as pl


def kernel(inputs, mask):
    raise NotImplementedError("write your pallas kernel here")


# speedup vs baseline: 15.9444x; 15.9444x over previous
import functools

import jax
import jax.numpy as jnp
from jax import lax
from jax.experimental import pallas as pl
from jax.experimental.pallas import tpu as pltpu
from jax.experimental.pallas import tpu_sc as plsc

L = 16
ROW = 8192
NCHUNK = ROW // L
NROWS = 64 * 32
NWORK = 32
ROWS_PER_W = NROWS // NWORK
CAND_MAX = 2048
NEG = -1e30


def _sc_body(x_hbm, out_hbm, rowbuf, outbuf, candi, candv):
    wid = lax.axis_index("s") * 2 + lax.axis_index("c")
    iota = lax.iota(jnp.int32, L)
    zeros = jnp.zeros((L,), jnp.float32)

    def row_body(r, _):
        row = wid * ROWS_PER_W + r
        pltpu.sync_copy(x_hbm.at[row], rowbuf)

        def abody(c, mx):
            v = rowbuf[pl.ds(c * L, L)]
            outbuf[pl.ds(c * L, L)] = zeros
            return jnp.maximum(mx, v)

        mx = lax.fori_loop(0, NCHUNK, abody, jnp.full((L,), NEG, jnp.float32))
        bound = jnp.full((L,), jnp.max(mx) - jnp.float32(1.0), jnp.float32)

        def bbody(c, off):
            v = rowbuf[pl.ds(c * L, L)]
            m = v > bound
            idx = c * L + iota
            plsc.store_compressed(candi.at[pl.ds(off, L)], idx, mask=m)
            cnt = jnp.sum(jnp.where(m, 1, 0))
            return jnp.minimum(off + cnt, CAND_MAX)

        k = lax.fori_loop(0, NCHUNK, bbody, 0)
        nk = (k + L - 1) // L

        def gbody(j, _):
            ok = j * L + iota < k
            idxs = jnp.where(ok, candi[pl.ds(j * L, L)], 0)
            v = plsc.load_gather(rowbuf, [idxs])
            candv[pl.ds(j * L, L)] = jnp.where(ok, v, NEG)
            return 0

        lax.fori_loop(0, nk, gbody, 0)

        def cond(carry):
            i, _, changed = carry
            return changed & (i < 300)

        def step(carry):
            i, tau, _ = carry

            def ibody(j, acc):
                s, c = acc
                v = candv[pl.ds(j * L, L)]
                m = v > tau
                return s + jnp.where(m, v, 0.0), c + jnp.where(m, 1, 0)

            s, c = lax.fori_loop(0, nk, ibody, (zeros, jnp.zeros((L,), jnp.int32)))
            csum = jnp.maximum(jnp.sum(c), 1).astype(jnp.float32)
            ssum = jnp.sum(s)
            tau_new = (jnp.full((L,), ssum) - jnp.float32(1.0)) / jnp.full((L,), csum)
            return i + 1, tau_new, jnp.any(tau_new != tau)

        _, tau, _ = lax.while_loop(cond, step, (0, bound, True))

        def sbody(j, _):
            ok = j * L + iota < k
            idxs = jnp.where(ok, candi[pl.ds(j * L, L)], 0)
            w = jnp.maximum(candv[pl.ds(j * L, L)] - tau, 0.0)
            plsc.store_scatter(outbuf, [idxs], w, mask=ok)
            return 0

        lax.fori_loop(0, nk, sbody, 0)

        pltpu.sync_copy(outbuf, out_hbm.at[row])
        return 0

    lax.fori_loop(0, ROWS_PER_W, row_body, 0)


@jax.jit
def _sparsemax_sc(x):
    f = pl.kernel(
        _sc_body,
        out_type=jax.ShapeDtypeStruct((NROWS, ROW), jnp.float32),
        mesh=plsc.VectorSubcoreMesh(core_axis_name="c", subcore_axis_name="s"),
        scratch_types=[
            pltpu.VMEM((ROW,), jnp.float32),
            pltpu.VMEM((ROW,), jnp.float32),
            pltpu.VMEM((CAND_MAX + L,), jnp.int32),
            pltpu.VMEM((CAND_MAX + L,), jnp.float32),
        ],
        compiler_params=pltpu.CompilerParams(needs_layout_passes=False),
    )
    return f(x)


def kernel(inputs, mask):
    del mask
    x = inputs.reshape(NROWS, ROW)
    return _sparsemax_sc(x).reshape(inputs.shape)


# speedup vs baseline: 17.9215x; 1.1240x over previous
import functools

import jax
import jax.numpy as jnp
from jax import lax
from jax.experimental import pallas as pl
from jax.experimental.pallas import tpu as pltpu
from jax.experimental.pallas import tpu_sc as plsc

L = 16
ROW = 8192
NCHUNK = ROW // L
NROWS = 64 * 32
NWORK = 32
ROWS_PER_W = NROWS // NWORK
CAND_MAX = 2048
NEG = -1e30


def _sc_body(x_hbm, out_hbm, rowbuf, outbuf, candi, candv):
    wid = lax.axis_index("s") * 2 + lax.axis_index("c")
    iota = lax.iota(jnp.int32, L)
    zeros = jnp.zeros((L,), jnp.float32)

    def row_body(r, _):
        row = wid * ROWS_PER_W + r
        pltpu.sync_copy(x_hbm.at[row], rowbuf)

        @plsc.parallel_loop(
            0, NCHUNK, unroll=8, carry=jnp.full((L,), NEG, jnp.float32)
        )
        def mx(c, acc):
            v = rowbuf[pl.ds(c * L, L)]
            outbuf[pl.ds(c * L, L)] = zeros
            return jnp.maximum(acc, v)

        bound = jnp.full((L,), jnp.max(mx) - jnp.float32(1.0), jnp.float32)

        @pl.loop(0, NCHUNK, init_carry=0, unroll=8)
        def k(c, off):
            v = rowbuf[pl.ds(c * L, L)]
            m = v > bound
            idx = c * L + iota
            plsc.store_compressed(candi.at[pl.ds(off, L)], idx, mask=m)
            cnt = jnp.sum(jnp.where(m, 1, 0))
            return jnp.minimum(off + cnt, CAND_MAX)
        nk = (k + L - 1) // L

        def gbody(j, _):
            ok = j * L + iota < k
            idxs = jnp.where(ok, candi[pl.ds(j * L, L)], 0)
            v = plsc.load_gather(rowbuf, [idxs])
            candv[pl.ds(j * L, L)] = jnp.where(ok, v, NEG)
            return 0

        lax.fori_loop(0, nk, gbody, 0)

        def cond(carry):
            i, _, changed = carry
            return changed & (i < 300)

        def step(carry):
            i, tau, _ = carry

            def ibody(j, acc):
                s, c = acc
                v = candv[pl.ds(j * L, L)]
                m = v > tau
                return s + jnp.where(m, v, 0.0), c + jnp.where(m, 1, 0)

            s, c = lax.fori_loop(0, nk, ibody, (zeros, jnp.zeros((L,), jnp.int32)))
            csum = jnp.maximum(jnp.sum(c), 1).astype(jnp.float32)
            ssum = jnp.sum(s)
            tau_new = (jnp.full((L,), ssum) - jnp.float32(1.0)) / jnp.full((L,), csum)
            return i + 1, tau_new, jnp.any(tau_new != tau)

        _, tau, _ = lax.while_loop(cond, step, (0, bound, True))

        def sbody(j, _):
            ok = j * L + iota < k
            idxs = jnp.where(ok, candi[pl.ds(j * L, L)], 0)
            w = jnp.maximum(candv[pl.ds(j * L, L)] - tau, 0.0)
            plsc.store_scatter(outbuf, [idxs], w, mask=ok)
            return 0

        lax.fori_loop(0, nk, sbody, 0)

        pltpu.sync_copy(outbuf, out_hbm.at[row])
        return 0

    lax.fori_loop(0, ROWS_PER_W, row_body, 0)


@jax.jit
def _sparsemax_sc(x):
    f = pl.kernel(
        _sc_body,
        out_type=jax.ShapeDtypeStruct((NROWS, ROW), jnp.float32),
        mesh=plsc.VectorSubcoreMesh(core_axis_name="c", subcore_axis_name="s"),
        scratch_types=[
            pltpu.VMEM((ROW,), jnp.float32),
            pltpu.VMEM((ROW,), jnp.float32),
            pltpu.VMEM((CAND_MAX + L,), jnp.int32),
            pltpu.VMEM((CAND_MAX + L,), jnp.float32),
        ],
        compiler_params=pltpu.CompilerParams(needs_layout_passes=False),
    )
    return f(x)


def kernel(inputs, mask):
    del mask
    x = inputs.reshape(NROWS, ROW)
    return _sparsemax_sc(x).reshape(inputs.shape)
